# Initial kernel scaffold; baseline (speedup 1.0000x reference)
import jax
import jax.numpy as jnp
from jax.experimental import pallas as pl


def kernel(query, memory_embeddings, memory_som_masks, W, b, k):
    raise NotImplementedError("write your pallas kernel here")


# speedup vs baseline: 2.5496x; 2.5496x over previous
import functools

import jax
import jax.numpy as jnp
from jax import lax
from jax.experimental import pallas as pl
from jax.experimental.pallas import tpu as pltpu
from jax.experimental.pallas import tpu_sc as plsc

Q_SIZE = 1024
K_SIZE = 100000
D_SIZE = 128
MASKW = 200
MASKW_PAD = 256
K_PAD = 100352
KB = 2048
NKB = K_PAD // KB
BMIN = 128
NB = K_PAD // BMIN
NB_PAD = 896
QT = 256
NQT = Q_SIZE // QT
TOPK_N = 16
NCAND = TOPK_N * BMIN
BIG_F = 1e30
BIG_I = 2**30

NC = 2
NS = 16
NW = NC * NS


def _rowsumsq(h):
    x = h * h
    acc = x[:, 0:8]
    for i in range(1, 16):
        acc = acc + x[:, 8 * i:8 * (i + 1)]
    w = 4
    while w >= 1:
        acc = acc[:, :w] + acc[:, w:2 * w]
        w //= 2
    return acc


def _proj_body(x_ref, w_ref, b_ref, h_ref, n_ref):
    h = jnp.tanh(
        lax.dot_general(x_ref[...], w_ref[...], (((1,), (0,)), ((), ())),
                        preferred_element_type=jnp.float32) + b_ref[...])
    norm = jnp.sqrt(_rowsumsq(h))
    scale = jnp.where(norm > 0.95, 0.95 / norm, jnp.float32(1.0))
    h2 = h * scale
    h_ref[...] = h2
    n_ref[...] = _rowsumsq(h2)


def _project(x, w, b, rows_per_step):
    n = x.shape[0]
    grid = (n // rows_per_step,)
    return pl.pallas_call(
        _proj_body,
        grid=grid,
        in_specs=[
            pl.BlockSpec((rows_per_step, D_SIZE), lambda i: (i, 0)),
            pl.BlockSpec((D_SIZE, D_SIZE), lambda i: (0, 0)),
            pl.BlockSpec((1, D_SIZE), lambda i: (0, 0)),
        ],
        out_specs=[
            pl.BlockSpec((rows_per_step, D_SIZE), lambda i: (i, 0)),
            pl.BlockSpec((rows_per_step, 1), lambda i: (i, 0)),
        ],
        out_shape=[
            jax.ShapeDtypeStruct((n, D_SIZE), jnp.float32),
            jax.ShapeDtypeStruct((n, 1), jnp.float32),
        ],
    )(x, w, b)


def _score_body(qh_ref, qn_ref, mh_ref, yn_ref, s_ref, bm_ref):
    xy = lax.dot_general(qh_ref[...], mh_ref[...], (((1,), (1,)), ((), ())),
                         preferred_element_type=jnp.float32)
    qn = qn_ref[...]
    yn = yn_ref[...]
    num = jnp.maximum(qn + yn - 2.0 * xy, 0.0)
    den = (1.0 - qn) * (1.0 - yn)
    s = (2.0 * num) / (den + 1e-08)
    col = pl.program_id(1) * KB + lax.broadcasted_iota(jnp.int32, s.shape, 1)
    s = jnp.where(col >= K_SIZE, BIG_F, s)
    s_ref[...] = s
    bm_ref[...] = jnp.min(s.reshape(QT, KB // BMIN, BMIN), axis=2
                          ).reshape(1, 1, QT, KB // BMIN)


def _scores(q_hyp, q_nsq, m_hyp, y_nsq_row):
    return pl.pallas_call(
        _score_body,
        grid=(NQT, NKB),
        in_specs=[
            pl.BlockSpec((QT, D_SIZE), lambda q, k: (q, 0)),
            pl.BlockSpec((QT, 1), lambda q, k: (q, 0)),
            pl.BlockSpec((KB, D_SIZE), lambda q, k: (k, 0)),
            pl.BlockSpec((1, KB), lambda q, k: (0, k)),
        ],
        out_specs=[
            pl.BlockSpec((QT, KB), lambda q, k: (q, k)),
            pl.BlockSpec((1, 1, QT, KB // BMIN), lambda q, k: (q, k, 0, 0)),
        ],
        out_shape=[
            jax.ShapeDtypeStruct((Q_SIZE, K_PAD), jnp.float32),
            jax.ShapeDtypeStruct((NQT, NKB, QT, KB // BMIN), jnp.float32),
        ],
    )(q_hyp, q_nsq, m_hyp, y_nsq_row)


def _blocktop_body(bm_ref, out_ref):
    m = bm_ref[...]
    col = lax.broadcasted_iota(jnp.int32, m.shape, 1)
    m = jnp.where(col >= NB, BIG_F, m)
    ids = []
    for _ in range(TOPK_N):
        v = jnp.min(m, axis=1, keepdims=True)
        bi = jnp.min(jnp.where(m == v, col, BIG_I), axis=1, keepdims=True)
        ids.append(bi)
        m = jnp.where(col == bi, BIG_F, m)
    blk = jnp.concatenate(ids, axis=1)
    out_cols = []
    for _ in range(TOPK_N):
        mn = jnp.min(blk, axis=1, keepdims=True)
        out_cols.append(mn)
        blk = jnp.where(blk == mn, BIG_I, blk)
    out_ref[...] = jnp.concatenate(out_cols, axis=1)


def _blocktop(bm_pad):
    return pl.pallas_call(
        _blocktop_body,
        grid=(NQT,),
        in_specs=[pl.BlockSpec((QT, NB_PAD), lambda q: (q, 0))],
        out_specs=pl.BlockSpec((QT, TOPK_N), lambda q: (q, 0)),
        out_shape=jax.ShapeDtypeStruct((Q_SIZE, TOPK_N), jnp.int32),
    )(bm_pad)


def _final_body(cand_ref, blk_ref, w_ref, idx_ref):
    s = cand_ref[...]
    blk = blk_ref[...]
    gk = (blk[:, :, None] * BMIN
          + lax.broadcasted_iota(jnp.int32, (QT, TOPK_N, BMIN), 2)
          ).reshape(QT, NCAND)
    z = jnp.maximum(1.0 + s, 1.0 + 1e-6)
    sm1 = jnp.sqrt(z - 1.0)
    d = jnp.log1p(sm1 * (sm1 + jnp.sqrt(z + 1.0)))
    vals, idxs = [], []
    for _ in range(TOPK_N):
        v = jnp.min(d, axis=1, keepdims=True)
        gi = jnp.min(jnp.where(d == v, gk, BIG_I), axis=1, keepdims=True)
        vals.append(v)
        idxs.append(gi)
        d = jnp.where((d == v) & (gk == gi), BIG_F, d)
    d16 = jnp.concatenate(vals, axis=1)
    i16 = jnp.concatenate(idxs, axis=1)
    e = jnp.exp(d16[:, 0:1] - d16)
    acc = e
    w = 8
    while w >= 1:
        acc = acc[:, :w] + acc[:, w:2 * w]
        w //= 2
    w_ref[...] = e / acc
    idx_ref[...] = i16


def _final(cand, blk_sorted):
    return pl.pallas_call(
        _final_body,
        grid=(NQT,),
        in_specs=[
            pl.BlockSpec((QT, NCAND), lambda q: (q, 0)),
            pl.BlockSpec((QT, TOPK_N), lambda q: (q, 0)),
        ],
        out_specs=[
            pl.BlockSpec((QT, TOPK_N), lambda q: (q, 0)),
            pl.BlockSpec((QT, TOPK_N), lambda q: (q, 0)),
        ],
        out_shape=[
            jax.ShapeDtypeStruct((Q_SIZE, TOPK_N), jnp.float32),
            jax.ShapeDtypeStruct((Q_SIZE, TOPK_N), jnp.int32),
        ],
    )(cand, blk_sorted)


def _sc_gather_candidates(blk_sorted, score_rows):
    qp = Q_SIZE // NW

    mesh = plsc.VectorSubcoreMesh(core_axis_name="c", subcore_axis_name="s")

    @functools.partial(
        pl.kernel, mesh=mesh,
        out_type=jax.ShapeDtypeStruct((Q_SIZE, TOPK_N, BMIN), jnp.float32),
        scratch_types=[
            pltpu.VMEM((TOPK_N,), jnp.int32),
            pltpu.VMEM((TOPK_N, BMIN), jnp.float32),
            pltpu.SemaphoreType.DMA,
        ],
    )
    def k(blk_hbm, srows_hbm, out_hbm, idx_v, rows_v, sem):
        wid = lax.axis_index("s") * NC + lax.axis_index("c")

        def body(i, carry):
            q = wid * qp + i
            pltpu.sync_copy(blk_hbm.at[q], idx_v)
            idx_v[...] = idx_v[...] + q * NB
            pltpu.async_copy(srows_hbm.at[idx_v], rows_v, sem).wait()
            pltpu.sync_copy(rows_v, out_hbm.at[q])
            return carry

        lax.fori_loop(0, qp, body, 0)

    return k(blk_sorted, score_rows)


def _sc_gather_masks(idx_flat, masks_pad):
    rows_total = Q_SIZE * TOPK_N
    rp = rows_total // NW
    chunk = 128

    mesh = plsc.VectorSubcoreMesh(core_axis_name="c", subcore_axis_name="s")

    @functools.partial(
        pl.kernel, mesh=mesh,
        out_type=jax.ShapeDtypeStruct((rows_total, MASKW_PAD), jnp.float32),
        scratch_types=[
            pltpu.VMEM((rp,), jnp.int32),
            pltpu.VMEM((chunk, MASKW_PAD), jnp.float32),
            pltpu.SemaphoreType.DMA,
        ],
    )
    def k(idx_hbm, masks_hbm, out_hbm, idx_v, rows_v, sem):
        wid = lax.axis_index("s") * NC + lax.axis_index("c")
        base = wid * rp
        pltpu.sync_copy(idx_hbm.at[pl.ds(base, rp)], idx_v)
        for j in range(rp // chunk):
            pltpu.async_copy(
                masks_hbm.at[idx_v.at[pl.ds(j * chunk, chunk)]],
                rows_v, sem).wait()
            pltpu.sync_copy(rows_v, out_hbm.at[pl.ds(base + j * chunk, chunk)])

    return k(idx_flat, masks_pad)


def kernel(query, memory_embeddings, memory_som_masks, W, b, k):
    del k
    b_row = b.reshape(1, D_SIZE)
    mem_pad = jnp.pad(memory_embeddings, ((0, K_PAD - K_SIZE), (0, 0)))
    masks_pad = jnp.pad(memory_som_masks, ((0, 0), (0, MASKW_PAD - MASKW)))

    q_hyp, q_nsq = _project(query, W, b_row, Q_SIZE)
    m_hyp, m_nsq = _project(mem_pad, W, b_row, 1024)
    y_nsq_row = m_nsq.reshape(1, K_PAD)

    scores, bm4 = _scores(q_hyp, q_nsq, m_hyp, y_nsq_row)
    bm = jnp.transpose(bm4, (0, 2, 1, 3)).reshape(Q_SIZE, NB)
    bm_pad = jnp.pad(bm, ((0, 0), (0, NB_PAD - NB)))
    blk_sorted = _blocktop(bm_pad)

    score_rows = scores.reshape(Q_SIZE * NB, BMIN)
    cand = _sc_gather_candidates(blk_sorted, score_rows)

    weights, top_idx = _final(cand.reshape(Q_SIZE, NCAND), blk_sorted)

    som_rows = _sc_gather_masks(top_idx.reshape(Q_SIZE * TOPK_N), masks_pad)
    som_hints = som_rows[:, :MASKW].reshape(Q_SIZE, TOPK_N, MASKW)
    return (weights, som_hints)


# speedup vs baseline: 3.0724x; 1.2051x over previous
import functools

import jax
import jax.numpy as jnp
from jax import lax
from jax.experimental import pallas as pl
from jax.experimental.pallas import tpu as pltpu
from jax.experimental.pallas import tpu_sc as plsc

Q_SIZE = 1024
K_SIZE = 100000
D_SIZE = 128
MASKW = 200
MASKW_PAD = 256
K_PAD = 100352
KB = 2048
NKB = K_PAD // KB
BMIN = 128
NB = K_PAD // BMIN
NB_PAD = 896
QT = 256
NQT = Q_SIZE // QT
TOPK_N = 16
NCAND = TOPK_N * BMIN
BIG_F = 1e30
BIG_I = 2**30

NC = 2
NS = 16
NW = NC * NS


def _rowsumsq(h):
    x = h * h
    acc = x[:, 0:8]
    for i in range(1, 16):
        acc = acc + x[:, 8 * i:8 * (i + 1)]
    w = 4
    while w >= 1:
        acc = acc[:, :w] + acc[:, w:2 * w]
        w //= 2
    return acc


def _proj_body(x_ref, w_ref, b_ref, h_ref, n_ref):
    h = jnp.tanh(
        lax.dot_general(x_ref[...], w_ref[...], (((1,), (0,)), ((), ())),
                        preferred_element_type=jnp.float32) + b_ref[...])
    norm = jnp.sqrt(_rowsumsq(h))
    scale = jnp.where(norm > 0.95, 0.95 / norm, jnp.float32(1.0))
    h2 = h * scale
    h_ref[...] = h2
    n_ref[...] = _rowsumsq(h2)


def _project(x, w, b, rows_per_step):
    n = x.shape[0]
    grid = (n // rows_per_step,)
    return pl.pallas_call(
        _proj_body,
        grid=grid,
        in_specs=[
            pl.BlockSpec((rows_per_step, D_SIZE), lambda i: (i, 0)),
            pl.BlockSpec((D_SIZE, D_SIZE), lambda i: (0, 0)),
            pl.BlockSpec((1, D_SIZE), lambda i: (0, 0)),
        ],
        out_specs=[
            pl.BlockSpec((rows_per_step, D_SIZE), lambda i: (i, 0)),
            pl.BlockSpec((rows_per_step, 1), lambda i: (i, 0)),
        ],
        out_shape=[
            jax.ShapeDtypeStruct((n, D_SIZE), jnp.float32),
            jax.ShapeDtypeStruct((n, 1), jnp.float32),
        ],
    )(x, w, b)


def _score_body(qh_ref, qn_ref, mh_ref, yn_ref, s_ref, bm_ref):
    xy = lax.dot_general(qh_ref[...], mh_ref[...], (((1,), (1,)), ((), ())),
                         preferred_element_type=jnp.float32)
    qn = qn_ref[...]
    yn = yn_ref[...]
    num = jnp.maximum(qn + yn - 2.0 * xy, 0.0)
    den = (1.0 - qn) * (1.0 - yn)
    s = (2.0 * num) / (den + 1e-08)
    ragged = pl.program_id(1) == (NKB - 1)

    @pl.when(ragged)
    def _():
        col = (NKB - 1) * KB + lax.broadcasted_iota(jnp.int32, s.shape, 1)
        sm = jnp.where(col >= K_SIZE, BIG_F, s)
        s_ref[...] = sm
        bm_ref[...] = jnp.min(sm.reshape(QT, KB // BMIN, BMIN), axis=2
                              ).reshape(1, 1, QT, KB // BMIN)

    @pl.when(jnp.logical_not(ragged))
    def _():
        s_ref[...] = s
        bm_ref[...] = jnp.min(s.reshape(QT, KB // BMIN, BMIN), axis=2
                              ).reshape(1, 1, QT, KB // BMIN)


def _scores(q_hyp, q_nsq, m_hyp, y_nsq_row):
    return pl.pallas_call(
        _score_body,
        grid=(NQT, NKB),
        in_specs=[
            pl.BlockSpec((QT, D_SIZE), lambda q, k: (q, 0)),
            pl.BlockSpec((QT, 1), lambda q, k: (q, 0)),
            pl.BlockSpec((KB, D_SIZE), lambda q, k: (k, 0)),
            pl.BlockSpec((1, KB), lambda q, k: (0, k)),
        ],
        out_specs=[
            pl.BlockSpec((QT, KB), lambda q, k: (q, k)),
            pl.BlockSpec((1, 1, QT, KB // BMIN), lambda q, k: (q, k, 0, 0)),
        ],
        out_shape=[
            jax.ShapeDtypeStruct((Q_SIZE, K_PAD), jnp.float32),
            jax.ShapeDtypeStruct((NQT, NKB, QT, KB // BMIN), jnp.float32),
        ],
    )(q_hyp, q_nsq, m_hyp, y_nsq_row)


def _blocktop_body(bm_ref, out_ref):
    m = bm_ref[...]
    col = lax.broadcasted_iota(jnp.int32, m.shape, 1)
    m = jnp.where(col >= NB, BIG_F, m)
    ids = []
    for _ in range(TOPK_N):
        v = jnp.min(m, axis=1, keepdims=True)
        bi = jnp.min(jnp.where(m == v, col, BIG_I), axis=1, keepdims=True)
        ids.append(bi)
        m = jnp.where(col == bi, BIG_F, m)
    blk = jnp.concatenate(ids, axis=1)
    out_cols = []
    for _ in range(TOPK_N):
        mn = jnp.min(blk, axis=1, keepdims=True)
        out_cols.append(mn)
        blk = jnp.where(blk == mn, BIG_I, blk)
    out_ref[...] = jnp.concatenate(out_cols, axis=1)


def _blocktop(bm_pad):
    return pl.pallas_call(
        _blocktop_body,
        grid=(NQT,),
        in_specs=[pl.BlockSpec((QT, NB_PAD), lambda q: (q, 0))],
        out_specs=pl.BlockSpec((QT, TOPK_N), lambda q: (q, 0)),
        out_shape=jax.ShapeDtypeStruct((Q_SIZE, TOPK_N), jnp.int32),
    )(bm_pad)


def _final_body(cand_ref, blk_ref, w_ref, idx_ref):
    s = cand_ref[...]
    blk = blk_ref[...]
    gk = (blk[:, :, None] * BMIN
          + lax.broadcasted_iota(jnp.int32, (QT, TOPK_N, BMIN), 2)
          ).reshape(QT, NCAND)
    z = jnp.maximum(1.0 + s, 1.0 + 1e-6)
    sm1 = jnp.sqrt(z - 1.0)
    d = jnp.log1p(sm1 * (sm1 + jnp.sqrt(z + 1.0)))
    vals, idxs = [], []
    for _ in range(TOPK_N):
        v = jnp.min(d, axis=1, keepdims=True)
        gi = jnp.min(jnp.where(d == v, gk, BIG_I), axis=1, keepdims=True)
        vals.append(v)
        idxs.append(gi)
        d = jnp.where((d == v) & (gk == gi), BIG_F, d)
    d16 = jnp.concatenate(vals, axis=1)
    i16 = jnp.concatenate(idxs, axis=1)
    e = jnp.exp(d16[:, 0:1] - d16)
    acc = e
    w = 8
    while w >= 1:
        acc = acc[:, :w] + acc[:, w:2 * w]
        w //= 2
    w_ref[...] = e / acc
    idx_ref[...] = i16


def _final(cand, blk_sorted):
    return pl.pallas_call(
        _final_body,
        grid=(NQT,),
        in_specs=[
            pl.BlockSpec((QT, NCAND), lambda q: (q, 0)),
            pl.BlockSpec((QT, TOPK_N), lambda q: (q, 0)),
        ],
        out_specs=[
            pl.BlockSpec((QT, TOPK_N), lambda q: (q, 0)),
            pl.BlockSpec((QT, TOPK_N), lambda q: (q, 0)),
        ],
        out_shape=[
            jax.ShapeDtypeStruct((Q_SIZE, TOPK_N), jnp.float32),
            jax.ShapeDtypeStruct((Q_SIZE, TOPK_N), jnp.int32),
        ],
    )(cand, blk_sorted)


def _sc_gather_candidates(blk_flat, score_rows):
    qp = Q_SIZE // NW
    rp = qp * TOPK_N
    chunk = 128

    mesh = plsc.VectorSubcoreMesh(core_axis_name="c", subcore_axis_name="s")

    @functools.partial(
        pl.kernel, mesh=mesh,
        out_type=jax.ShapeDtypeStruct((Q_SIZE * TOPK_N, BMIN), jnp.float32),
        scratch_types=[
            pltpu.VMEM((rp,), jnp.int32),
            pltpu.VMEM((chunk, BMIN), jnp.float32),
            pltpu.SemaphoreType.DMA,
        ],
    )
    def k(blk_hbm, srows_hbm, out_hbm, idx_v, rows_v, sem):
        wid = lax.axis_index("s") * NC + lax.axis_index("c")
        base = wid * rp
        pltpu.sync_copy(blk_hbm.at[pl.ds(base, rp)], idx_v)
        for j in range(qp):
            sl = pl.ds(TOPK_N * j, TOPK_N)
            idx_v[sl] = idx_v[sl] + (wid * qp + j) * NB
        for c in range(rp // chunk):
            pltpu.async_copy(
                srows_hbm.at[idx_v.at[pl.ds(c * chunk, chunk)]],
                rows_v, sem).wait()
            pltpu.sync_copy(rows_v, out_hbm.at[pl.ds(base + c * chunk, chunk)])

    return k(blk_flat, score_rows)


def _padmask_body(x_ref, o_ref):
    x = x_ref[...]
    o_ref[...] = jnp.concatenate(
        [x, jnp.zeros((x.shape[0], MASKW_PAD - MASKW), jnp.float32)], axis=1)


def _pad_masks(masks):
    rows = 2000
    return pl.pallas_call(
        _padmask_body,
        grid=(K_SIZE // rows,),
        in_specs=[pl.BlockSpec((rows, MASKW), lambda i: (i, 0))],
        out_specs=pl.BlockSpec((rows, MASKW_PAD), lambda i: (i, 0)),
        out_shape=jax.ShapeDtypeStruct((K_SIZE, MASKW_PAD), jnp.float32),
    )(masks)


def _sc_gather_masks(idx_flat, masks_pad):
    rows_total = Q_SIZE * TOPK_N
    rp = rows_total // NW
    chunk = 128

    mesh = plsc.VectorSubcoreMesh(core_axis_name="c", subcore_axis_name="s")

    @functools.partial(
        pl.kernel, mesh=mesh,
        out_type=jax.ShapeDtypeStruct((rows_total, MASKW_PAD), jnp.float32),
        scratch_types=[
            pltpu.VMEM((rp,), jnp.int32),
            pltpu.VMEM((chunk, MASKW_PAD), jnp.float32),
            pltpu.SemaphoreType.DMA,
        ],
    )
    def k(idx_hbm, masks_hbm, out_hbm, idx_v, rows_v, sem):
        wid = lax.axis_index("s") * NC + lax.axis_index("c")
        base = wid * rp
        pltpu.sync_copy(idx_hbm.at[pl.ds(base, rp)], idx_v)
        for j in range(rp // chunk):
            pltpu.async_copy(
                masks_hbm.at[idx_v.at[pl.ds(j * chunk, chunk)]],
                rows_v, sem).wait()
            pltpu.sync_copy(rows_v, out_hbm.at[pl.ds(base + j * chunk, chunk)])

    return k(idx_flat, masks_pad)


def kernel(query, memory_embeddings, memory_som_masks, W, b, k):
    del k
    b_row = b.reshape(1, D_SIZE)
    mem_pad = jnp.pad(memory_embeddings, ((0, K_PAD - K_SIZE), (0, 0)))
    masks_pad = _pad_masks(memory_som_masks)

    q_hyp, q_nsq = _project(query, W, b_row, Q_SIZE)
    m_hyp, m_nsq = _project(mem_pad, W, b_row, 1024)
    y_nsq_row = m_nsq.reshape(1, K_PAD)

    scores, bm4 = _scores(q_hyp, q_nsq, m_hyp, y_nsq_row)
    bm = jnp.transpose(bm4, (0, 2, 1, 3)).reshape(Q_SIZE, NB)
    bm_pad = jnp.pad(bm, ((0, 0), (0, NB_PAD - NB)))
    blk_sorted = _blocktop(bm_pad)

    score_rows = scores.reshape(Q_SIZE * NB, BMIN)
    cand = _sc_gather_candidates(blk_sorted.reshape(Q_SIZE * TOPK_N),
                                 score_rows)

    weights, top_idx = _final(cand.reshape(Q_SIZE, NCAND), blk_sorted)

    som_rows = _sc_gather_masks(top_idx.reshape(Q_SIZE * TOPK_N), masks_pad)
    som_hints = som_rows[:, :MASKW].reshape(Q_SIZE, TOPK_N, MASKW)
    return (weights, som_hints)


# speedup vs baseline: 3.1954x; 1.0400x over previous
import functools

import jax
import jax.numpy as jnp
from jax import lax
from jax.experimental import pallas as pl
from jax.experimental.pallas import tpu as pltpu
from jax.experimental.pallas import tpu_sc as plsc

Q_SIZE = 1024
K_SIZE = 100000
D_SIZE = 128
MASKW = 200
MASKW_PAD = 256
K_PAD = 100352
KB = 3584
NKB = K_PAD // KB
BMIN = 128
NB = K_PAD // BMIN
NB_PAD = 896
QT = 256
NQT = Q_SIZE // QT
QTA = 512
NQTA = Q_SIZE // QTA
TOPK_N = 16
NCAND = TOPK_N * BMIN
BIG_F = 1e30
BIG_I = 2**30

NC = 2
NS = 16
NW = NC * NS


def _rowsumsq(h):
    x = h * h
    acc = x[:, 0:8]
    for i in range(1, 16):
        acc = acc + x[:, 8 * i:8 * (i + 1)]
    w = 4
    while w >= 1:
        acc = acc[:, :w] + acc[:, w:2 * w]
        w //= 2
    return acc


def _proj_body(x_ref, w_ref, b_ref, h_ref, n_ref):
    h = jnp.tanh(
        lax.dot_general(x_ref[...], w_ref[...], (((1,), (0,)), ((), ())),
                        preferred_element_type=jnp.float32) + b_ref[...])
    norm = jnp.sqrt(_rowsumsq(h))
    scale = jnp.where(norm > 0.95, 0.95 / norm, jnp.float32(1.0))
    h2 = h * scale
    h_ref[...] = h2
    n_ref[...] = _rowsumsq(h2)


def _project(x, w, b, rows_per_step):
    n = x.shape[0]
    grid = (n // rows_per_step,)
    return pl.pallas_call(
        _proj_body,
        grid=grid,
        in_specs=[
            pl.BlockSpec((rows_per_step, D_SIZE), lambda i: (i, 0)),
            pl.BlockSpec((D_SIZE, D_SIZE), lambda i: (0, 0)),
            pl.BlockSpec((1, D_SIZE), lambda i: (0, 0)),
        ],
        out_specs=[
            pl.BlockSpec((rows_per_step, D_SIZE), lambda i: (i, 0)),
            pl.BlockSpec((rows_per_step, 1), lambda i: (i, 0)),
        ],
        out_shape=[
            jax.ShapeDtypeStruct((n, D_SIZE), jnp.float32),
            jax.ShapeDtypeStruct((n, 1), jnp.float32),
        ],
    )(x, w, b)


def _score_body(qh_ref, qn_ref, mh_ref, yn_ref, s_ref, bm_ref):
    xy = lax.dot_general(qh_ref[...], mh_ref[...], (((1,), (1,)), ((), ())),
                         preferred_element_type=jnp.float32)
    qn = qn_ref[...]
    yn = yn_ref[...]
    num = jnp.maximum(qn + yn - 2.0 * xy, 0.0)
    den = (1.0 - qn) * (1.0 - yn)
    s = (2.0 * num) / (den + 1e-08)
    ragged = pl.program_id(1) == (NKB - 1)

    @pl.when(ragged)
    def _():
        col = (NKB - 1) * KB + lax.broadcasted_iota(jnp.int32, s.shape, 1)
        sm = jnp.where(col >= K_SIZE, BIG_F, s)
        s_ref[...] = sm
        bm_ref[...] = jnp.min(sm.reshape(QTA, KB // BMIN, BMIN), axis=2
                              ).reshape(1, 1, QTA, KB // BMIN)

    @pl.when(jnp.logical_not(ragged))
    def _():
        s_ref[...] = s
        bm_ref[...] = jnp.min(s.reshape(QTA, KB // BMIN, BMIN), axis=2
                              ).reshape(1, 1, QTA, KB // BMIN)


def _scores(q_hyp, q_nsq, m_hyp, y_nsq_row):
    return pl.pallas_call(
        _score_body,
        grid=(NQTA, NKB),
        in_specs=[
            pl.BlockSpec((QTA, D_SIZE), lambda q, k: (q, 0)),
            pl.BlockSpec((QTA, 1), lambda q, k: (q, 0)),
            pl.BlockSpec((KB, D_SIZE), lambda q, k: (k, 0)),
            pl.BlockSpec((1, KB), lambda q, k: (0, k)),
        ],
        out_specs=[
            pl.BlockSpec((QTA, KB), lambda q, k: (q, k)),
            pl.BlockSpec((1, 1, QTA, KB // BMIN), lambda q, k: (q, k, 0, 0)),
        ],
        out_shape=[
            jax.ShapeDtypeStruct((Q_SIZE, K_PAD), jnp.float32),
            jax.ShapeDtypeStruct((NQTA, NKB, QTA, KB // BMIN), jnp.float32),
        ],
    )(q_hyp, q_nsq, m_hyp, y_nsq_row)


def _blocktop_body(bm_ref, out_ref):
    m = bm_ref[...]
    col = lax.broadcasted_iota(jnp.int32, m.shape, 1)
    m = jnp.where(col >= NB, BIG_F, m)
    ids = []
    for _ in range(TOPK_N):
        v = jnp.min(m, axis=1, keepdims=True)
        bi = jnp.min(jnp.where(m == v, col, BIG_I), axis=1, keepdims=True)
        ids.append(bi)
        m = jnp.where(col == bi, BIG_F, m)
    blk = jnp.concatenate(ids, axis=1)
    out_cols = []
    for _ in range(TOPK_N):
        mn = jnp.min(blk, axis=1, keepdims=True)
        out_cols.append(mn)
        blk = jnp.where(blk == mn, BIG_I, blk)
    out_ref[...] = jnp.concatenate(out_cols, axis=1)


def _blocktop(bm_pad):
    return pl.pallas_call(
        _blocktop_body,
        grid=(NQT,),
        in_specs=[pl.BlockSpec((QT, NB_PAD), lambda q: (q, 0))],
        out_specs=pl.BlockSpec((QT, TOPK_N), lambda q: (q, 0)),
        out_shape=jax.ShapeDtypeStruct((Q_SIZE, TOPK_N), jnp.int32),
    )(bm_pad)


def _final_body(cand_ref, blk_ref, w_ref, idx_ref):
    s = cand_ref[...]
    blk = blk_ref[...]
    gk = (blk[:, :, None] * BMIN
          + lax.broadcasted_iota(jnp.int32, (QT, TOPK_N, BMIN), 2)
          ).reshape(QT, NCAND)
    z = jnp.maximum(1.0 + s, 1.0 + 1e-6)
    sm1 = jnp.sqrt(z - 1.0)
    d = jnp.log1p(sm1 * (sm1 + jnp.sqrt(z + 1.0)))
    vals, idxs = [], []
    for _ in range(TOPK_N):
        v = jnp.min(d, axis=1, keepdims=True)
        gi = jnp.min(jnp.where(d == v, gk, BIG_I), axis=1, keepdims=True)
        vals.append(v)
        idxs.append(gi)
        d = jnp.where(gk == gi, BIG_F, d)
    d16 = jnp.concatenate(vals, axis=1)
    i16 = jnp.concatenate(idxs, axis=1)
    e = jnp.exp(d16[:, 0:1] - d16)
    acc = e
    w = 8
    while w >= 1:
        acc = acc[:, :w] + acc[:, w:2 * w]
        w //= 2
    w_ref[...] = e / acc
    idx_ref[...] = i16


def _final(cand, blk_sorted):
    return pl.pallas_call(
        _final_body,
        grid=(NQT,),
        in_specs=[
            pl.BlockSpec((QT, NCAND), lambda q: (q, 0)),
            pl.BlockSpec((QT, TOPK_N), lambda q: (q, 0)),
        ],
        out_specs=[
            pl.BlockSpec((QT, TOPK_N), lambda q: (q, 0)),
            pl.BlockSpec((QT, TOPK_N), lambda q: (q, 0)),
        ],
        out_shape=[
            jax.ShapeDtypeStruct((Q_SIZE, TOPK_N), jnp.float32),
            jax.ShapeDtypeStruct((Q_SIZE, TOPK_N), jnp.int32),
        ],
    )(cand, blk_sorted)


def _sc_gather_candidates(blk_flat, score_rows):
    qp = Q_SIZE // NW
    rp = qp * TOPK_N
    chunk = 128

    mesh = plsc.VectorSubcoreMesh(core_axis_name="c", subcore_axis_name="s")

    @functools.partial(
        pl.kernel, mesh=mesh,
        out_type=jax.ShapeDtypeStruct((Q_SIZE * TOPK_N, BMIN), jnp.float32),
        scratch_types=[
            pltpu.VMEM((rp,), jnp.int32),
            pltpu.VMEM((chunk, BMIN), jnp.float32),
            pltpu.SemaphoreType.DMA,
        ],
    )
    def k(blk_hbm, srows_hbm, out_hbm, idx_v, rows_v, sem):
        wid = lax.axis_index("s") * NC + lax.axis_index("c")
        base = wid * rp
        pltpu.sync_copy(blk_hbm.at[pl.ds(base, rp)], idx_v)
        for j in range(qp):
            sl = pl.ds(TOPK_N * j, TOPK_N)
            idx_v[sl] = idx_v[sl] + (wid * qp + j) * NB
        for c in range(rp // chunk):
            pltpu.async_copy(
                srows_hbm.at[idx_v.at[pl.ds(c * chunk, chunk)]],
                rows_v, sem).wait()
            pltpu.sync_copy(rows_v, out_hbm.at[pl.ds(base + c * chunk, chunk)])

    return k(blk_flat, score_rows)


def _padmask_body(x_ref, o_ref):
    x = x_ref[...]
    o_ref[...] = jnp.concatenate(
        [x, jnp.zeros((x.shape[0], MASKW_PAD - MASKW), jnp.float32)], axis=1)


def _pad_masks(masks):
    rows = 2000
    return pl.pallas_call(
        _padmask_body,
        grid=(K_SIZE // rows,),
        in_specs=[pl.BlockSpec((rows, MASKW), lambda i: (i, 0))],
        out_specs=pl.BlockSpec((rows, MASKW_PAD), lambda i: (i, 0)),
        out_shape=jax.ShapeDtypeStruct((K_SIZE, MASKW_PAD), jnp.float32),
    )(masks)


def _sc_gather_masks(idx_flat, masks_pad):
    rows_total = Q_SIZE * TOPK_N
    rp = rows_total // NW
    chunk = 128

    mesh = plsc.VectorSubcoreMesh(core_axis_name="c", subcore_axis_name="s")

    @functools.partial(
        pl.kernel, mesh=mesh,
        out_type=jax.ShapeDtypeStruct((rows_total, MASKW_PAD), jnp.float32),
        scratch_types=[
            pltpu.VMEM((rp,), jnp.int32),
            pltpu.VMEM((chunk, MASKW_PAD), jnp.float32),
            pltpu.SemaphoreType.DMA,
        ],
    )
    def k(idx_hbm, masks_hbm, out_hbm, idx_v, rows_v, sem):
        wid = lax.axis_index("s") * NC + lax.axis_index("c")
        base = wid * rp
        pltpu.sync_copy(idx_hbm.at[pl.ds(base, rp)], idx_v)
        for j in range(rp // chunk):
            pltpu.async_copy(
                masks_hbm.at[idx_v.at[pl.ds(j * chunk, chunk)]],
                rows_v, sem).wait()
            pltpu.sync_copy(rows_v, out_hbm.at[pl.ds(base + j * chunk, chunk)])

    return k(idx_flat, masks_pad)


def kernel(query, memory_embeddings, memory_som_masks, W, b, k):
    del k
    b_row = b.reshape(1, D_SIZE)
    mem_pad = jnp.pad(memory_embeddings, ((0, K_PAD - K_SIZE), (0, 0)))
    masks_pad = _pad_masks(memory_som_masks)

    q_hyp, q_nsq = _project(query, W, b_row, Q_SIZE)
    m_hyp, m_nsq = _project(mem_pad, W, b_row, 1024)
    y_nsq_row = m_nsq.reshape(1, K_PAD)

    scores, bm4 = _scores(q_hyp, q_nsq, m_hyp, y_nsq_row)
    bm = jnp.transpose(bm4, (0, 2, 1, 3)).reshape(Q_SIZE, NB)
    bm_pad = jnp.pad(bm, ((0, 0), (0, NB_PAD - NB)))
    blk_sorted = _blocktop(bm_pad)

    score_rows = scores.reshape(Q_SIZE * NB, BMIN)
    cand = _sc_gather_candidates(blk_sorted.reshape(Q_SIZE * TOPK_N),
                                 score_rows)

    weights, top_idx = _final(cand.reshape(Q_SIZE, NCAND), blk_sorted)

    som_rows = _sc_gather_masks(top_idx.reshape(Q_SIZE * TOPK_N), masks_pad)
    som_hints = som_rows[:, :MASKW].reshape(Q_SIZE, TOPK_N, MASKW)
    return (weights, som_hints)


# speedup vs baseline: 3.2653x; 1.0219x over previous
import functools

import jax
import jax.numpy as jnp
from jax import lax
from jax.experimental import pallas as pl
from jax.experimental.pallas import tpu as pltpu
from jax.experimental.pallas import tpu_sc as plsc

Q_SIZE = 1024
K_SIZE = 100000
D_SIZE = 128
MASKW = 200
MASKW_PAD = 256
K_PAD = 100352
KB = 2048
NKB = K_PAD // KB
BMIN = 128
NB = K_PAD // BMIN
NB_PAD = 896
QT = 256
NQT = Q_SIZE // QT
QTA = 1024
NQTA = Q_SIZE // QTA
TOPK_N = 16
NCAND = TOPK_N * BMIN
BIG_F = 1e30
BIG_I = 2**30

NC = 2
NS = 16
NW = NC * NS


def _rowsumsq(h):
    x = h * h
    acc = x[:, 0:8]
    for i in range(1, 16):
        acc = acc + x[:, 8 * i:8 * (i + 1)]
    w = 4
    while w >= 1:
        acc = acc[:, :w] + acc[:, w:2 * w]
        w //= 2
    return acc


def _proj_body(x_ref, w_ref, b_ref, h_ref, n_ref):
    h = jnp.tanh(
        lax.dot_general(x_ref[...], w_ref[...], (((1,), (0,)), ((), ())),
                        preferred_element_type=jnp.float32) + b_ref[...])
    norm = jnp.sqrt(_rowsumsq(h))
    scale = jnp.where(norm > 0.95, 0.95 / norm, jnp.float32(1.0))
    h2 = h * scale
    h_ref[...] = h2
    n_ref[...] = _rowsumsq(h2)


def _project(x, w, b, rows_per_step):
    n = x.shape[0]
    grid = (n // rows_per_step,)
    return pl.pallas_call(
        _proj_body,
        grid=grid,
        in_specs=[
            pl.BlockSpec((rows_per_step, D_SIZE), lambda i: (i, 0)),
            pl.BlockSpec((D_SIZE, D_SIZE), lambda i: (0, 0)),
            pl.BlockSpec((1, D_SIZE), lambda i: (0, 0)),
        ],
        out_specs=[
            pl.BlockSpec((rows_per_step, D_SIZE), lambda i: (i, 0)),
            pl.BlockSpec((rows_per_step, 1), lambda i: (i, 0)),
        ],
        out_shape=[
            jax.ShapeDtypeStruct((n, D_SIZE), jnp.float32),
            jax.ShapeDtypeStruct((n, 1), jnp.float32),
        ],
    )(x, w, b)


def _score_body(qh_ref, qn_ref, mh_ref, yn_ref, s_ref, bm_ref):
    xy = lax.dot_general(qh_ref[...], mh_ref[...], (((1,), (1,)), ((), ())),
                         preferred_element_type=jnp.float32)
    qn = qn_ref[...]
    yn = yn_ref[...]
    num = jnp.maximum(qn + yn - 2.0 * xy, 0.0)
    den = (1.0 - qn) * (1.0 - yn)
    s = (2.0 * num) / (den + 1e-08)
    ragged = pl.program_id(1) == (NKB - 1)

    @pl.when(ragged)
    def _():
        col = (NKB - 1) * KB + lax.broadcasted_iota(jnp.int32, s.shape, 1)
        sm = jnp.where(col >= K_SIZE, BIG_F, s)
        s_ref[...] = sm
        bm_ref[...] = jnp.min(sm.reshape(QTA, KB // BMIN, BMIN), axis=2
                              ).reshape(1, 1, QTA, KB // BMIN)

    @pl.when(jnp.logical_not(ragged))
    def _():
        s_ref[...] = s
        bm_ref[...] = jnp.min(s.reshape(QTA, KB // BMIN, BMIN), axis=2
                              ).reshape(1, 1, QTA, KB // BMIN)


def _scores(q_hyp, q_nsq, m_hyp, y_nsq_row):
    return pl.pallas_call(
        _score_body,
        grid=(NQTA, NKB),
        in_specs=[
            pl.BlockSpec((QTA, D_SIZE), lambda q, k: (q, 0)),
            pl.BlockSpec((QTA, 1), lambda q, k: (q, 0)),
            pl.BlockSpec((KB, D_SIZE), lambda q, k: (k, 0)),
            pl.BlockSpec((1, KB), lambda q, k: (0, k)),
        ],
        out_specs=[
            pl.BlockSpec((QTA, KB), lambda q, k: (q, k)),
            pl.BlockSpec((1, 1, QTA, KB // BMIN), lambda q, k: (q, k, 0, 0)),
        ],
        out_shape=[
            jax.ShapeDtypeStruct((Q_SIZE, K_PAD), jnp.float32),
            jax.ShapeDtypeStruct((NQTA, NKB, QTA, KB // BMIN), jnp.float32),
        ],
    )(q_hyp, q_nsq, m_hyp, y_nsq_row)


def _blocktop_body(bm_ref, out_ref):
    m = bm_ref[...]
    col = lax.broadcasted_iota(jnp.int32, m.shape, 1)
    m = jnp.where(col >= NB, BIG_F, m)
    ids = []
    for _ in range(TOPK_N):
        v = jnp.min(m, axis=1, keepdims=True)
        bi = jnp.min(jnp.where(m == v, col, BIG_I), axis=1, keepdims=True)
        ids.append(bi)
        m = jnp.where(col == bi, BIG_F, m)
    blk = jnp.concatenate(ids, axis=1)
    out_cols = []
    for _ in range(TOPK_N):
        mn = jnp.min(blk, axis=1, keepdims=True)
        out_cols.append(mn)
        blk = jnp.where(blk == mn, BIG_I, blk)
    out_ref[...] = jnp.concatenate(out_cols, axis=1)


def _blocktop(bm_pad):
    return pl.pallas_call(
        _blocktop_body,
        grid=(NQT,),
        in_specs=[pl.BlockSpec((QT, NB_PAD), lambda q: (q, 0))],
        out_specs=pl.BlockSpec((QT, TOPK_N), lambda q: (q, 0)),
        out_shape=jax.ShapeDtypeStruct((Q_SIZE, TOPK_N), jnp.int32),
    )(bm_pad)


def _final_body(cand_ref, blk_ref, w_ref, idx_ref):
    s = cand_ref[...]
    blk = blk_ref[...]
    gk = (blk[:, :, None] * BMIN
          + lax.broadcasted_iota(jnp.int32, (QT, TOPK_N, BMIN), 2)
          ).reshape(QT, NCAND)
    z = jnp.maximum(1.0 + s, 1.0 + 1e-6)
    sm1 = jnp.sqrt(z - 1.0)
    d = jnp.log1p(sm1 * (sm1 + jnp.sqrt(z + 1.0)))
    vals, idxs = [], []
    for _ in range(TOPK_N):
        v = jnp.min(d, axis=1, keepdims=True)
        gi = jnp.min(jnp.where(d == v, gk, BIG_I), axis=1, keepdims=True)
        vals.append(v)
        idxs.append(gi)
        d = jnp.where(gk == gi, BIG_F, d)
    d16 = jnp.concatenate(vals, axis=1)
    i16 = jnp.concatenate(idxs, axis=1)
    e = jnp.exp(d16[:, 0:1] - d16)
    acc = e
    w = 8
    while w >= 1:
        acc = acc[:, :w] + acc[:, w:2 * w]
        w //= 2
    w_ref[...] = e / acc
    idx_ref[...] = i16


def _final(cand, blk_sorted):
    return pl.pallas_call(
        _final_body,
        grid=(NQT,),
        in_specs=[
            pl.BlockSpec((QT, NCAND), lambda q: (q, 0)),
            pl.BlockSpec((QT, TOPK_N), lambda q: (q, 0)),
        ],
        out_specs=[
            pl.BlockSpec((QT, TOPK_N), lambda q: (q, 0)),
            pl.BlockSpec((QT, TOPK_N), lambda q: (q, 0)),
        ],
        out_shape=[
            jax.ShapeDtypeStruct((Q_SIZE, TOPK_N), jnp.float32),
            jax.ShapeDtypeStruct((Q_SIZE, TOPK_N), jnp.int32),
        ],
    )(cand, blk_sorted)


def _sc_gather_candidates(blk_flat, score_rows):
    qp = Q_SIZE // NW
    rp = qp * TOPK_N
    chunk = 128

    mesh = plsc.VectorSubcoreMesh(core_axis_name="c", subcore_axis_name="s")

    @functools.partial(
        pl.kernel, mesh=mesh,
        out_type=jax.ShapeDtypeStruct((Q_SIZE * TOPK_N, BMIN), jnp.float32),
        scratch_types=[
            pltpu.VMEM((rp,), jnp.int32),
            pltpu.VMEM((chunk, BMIN), jnp.float32),
            pltpu.SemaphoreType.DMA,
        ],
    )
    def k(blk_hbm, srows_hbm, out_hbm, idx_v, rows_v, sem):
        wid = lax.axis_index("s") * NC + lax.axis_index("c")
        base = wid * rp
        pltpu.sync_copy(blk_hbm.at[pl.ds(base, rp)], idx_v)
        for j in range(qp):
            sl = pl.ds(TOPK_N * j, TOPK_N)
            idx_v[sl] = idx_v[sl] + (wid * qp + j) * NB
        for c in range(rp // chunk):
            pltpu.async_copy(
                srows_hbm.at[idx_v.at[pl.ds(c * chunk, chunk)]],
                rows_v, sem).wait()
            pltpu.sync_copy(rows_v, out_hbm.at[pl.ds(base + c * chunk, chunk)])

    return k(blk_flat, score_rows)


def _padmask_body(x_ref, o_ref):
    x = x_ref[...]
    o_ref[...] = jnp.concatenate(
        [x, jnp.zeros((x.shape[0], MASKW_PAD - MASKW), jnp.float32)], axis=1)


def _pad_masks(masks):
    rows = 2000
    return pl.pallas_call(
        _padmask_body,
        grid=(K_SIZE // rows,),
        in_specs=[pl.BlockSpec((rows, MASKW), lambda i: (i, 0))],
        out_specs=pl.BlockSpec((rows, MASKW_PAD), lambda i: (i, 0)),
        out_shape=jax.ShapeDtypeStruct((K_SIZE, MASKW_PAD), jnp.float32),
    )(masks)


def _sc_gather_masks(idx_flat, masks_pad):
    rows_total = Q_SIZE * TOPK_N
    rp = rows_total // NW
    chunk = 128

    mesh = plsc.VectorSubcoreMesh(core_axis_name="c", subcore_axis_name="s")

    @functools.partial(
        pl.kernel, mesh=mesh,
        out_type=jax.ShapeDtypeStruct((rows_total, MASKW_PAD), jnp.float32),
        scratch_types=[
            pltpu.VMEM((rp,), jnp.int32),
            pltpu.VMEM((chunk, MASKW_PAD), jnp.float32),
            pltpu.SemaphoreType.DMA,
        ],
    )
    def k(idx_hbm, masks_hbm, out_hbm, idx_v, rows_v, sem):
        wid = lax.axis_index("s") * NC + lax.axis_index("c")
        base = wid * rp
        pltpu.sync_copy(idx_hbm.at[pl.ds(base, rp)], idx_v)
        for j in range(rp // chunk):
            pltpu.async_copy(
                masks_hbm.at[idx_v.at[pl.ds(j * chunk, chunk)]],
                rows_v, sem).wait()
            pltpu.sync_copy(rows_v, out_hbm.at[pl.ds(base + j * chunk, chunk)])

    return k(idx_flat, masks_pad)


def kernel(query, memory_embeddings, memory_som_masks, W, b, k):
    del k
    b_row = b.reshape(1, D_SIZE)
    mem_pad = jnp.pad(memory_embeddings, ((0, K_PAD - K_SIZE), (0, 0)))
    masks_pad = _pad_masks(memory_som_masks)

    q_hyp, q_nsq = _project(query, W, b_row, Q_SIZE)
    m_hyp, m_nsq = _project(mem_pad, W, b_row, 1024)
    y_nsq_row = m_nsq.reshape(1, K_PAD)

    scores, bm4 = _scores(q_hyp, q_nsq, m_hyp, y_nsq_row)
    bm = jnp.transpose(bm4, (0, 2, 1, 3)).reshape(Q_SIZE, NB)
    bm_pad = jnp.pad(bm, ((0, 0), (0, NB_PAD - NB)))
    blk_sorted = _blocktop(bm_pad)

    score_rows = scores.reshape(Q_SIZE * NB, BMIN)
    cand = _sc_gather_candidates(blk_sorted.reshape(Q_SIZE * TOPK_N),
                                 score_rows)

    weights, top_idx = _final(cand.reshape(Q_SIZE, NCAND), blk_sorted)

    som_rows = _sc_gather_masks(top_idx.reshape(Q_SIZE * TOPK_N), masks_pad)
    som_hints = som_rows[:, :MASKW].reshape(Q_SIZE, TOPK_N, MASKW)
    return (weights, som_hints)
